# Initial kernel scaffold; baseline (speedup 1.0000x reference)
import jax
import jax.numpy as jnp
from jax.experimental import pallas as pl


def kernel(x, group_ids, conv_norm_w, conv_norm_b, conv_w, conv_b, attn_norm_w, attn_norm_b, in_proj_w, in_proj_b, out_proj_w, out_proj_b, w1, b1, w2, b2):
    raise NotImplementedError("write your pallas kernel here")


# speedup vs baseline: 2.1839x; 2.1839x over previous
import functools

import jax
import jax.numpy as jnp
from jax.experimental import pallas as pl
from jax.experimental.pallas import tpu as pltpu


_INTERPRET = False


def _gelu(z):
    return 0.5 * z * (1.0 + jax.lax.erf(z * 0.7071067811865476))


def _ln_pad_body(S, D, PADL, x_ref, cnw_ref, cnb_ref, o_ref):
    xv = x_ref[0]
    mu = jnp.mean(xv, axis=1, keepdims=True)
    var = jnp.mean((xv - mu) ** 2, axis=1, keepdims=True)
    h = (xv - mu) * jax.lax.rsqrt(var + 1e-5) * cnw_ref[0] + cnb_ref[0]
    o_ref[0, 0:PADL, :] = jnp.zeros((PADL, D), jnp.float32)
    o_ref[0, PADL:PADL + S, :] = h
    o_ref[0, PADL + S:, :] = jnp.zeros((o_ref.shape[1] - PADL - S, D),
                                       jnp.float32)


def _ln_pad(x, cnw, cnb, PADL, SP):
    B, S, D = x.shape
    return pl.pallas_call(
        functools.partial(_ln_pad_body, S, D, PADL),
        grid=(B,),
        in_specs=[
            pl.BlockSpec((1, S, D), lambda b: (b, 0, 0)),
            pl.BlockSpec((1, D), lambda b: (0, 0)),
            pl.BlockSpec((1, D), lambda b: (0, 0)),
        ],
        out_specs=pl.BlockSpec((1, SP, D), lambda b: (b, 0, 0)),
        out_shape=jax.ShapeDtypeStruct((B, SP, D), jnp.float32),
        interpret=_INTERPRET,
    )(x, cnw, cnb)


def _conv_body(S, D, OFF, KG, Q, hp_ref, wt_ref, cb_ref, o_ref):
    q = pl.program_id(2)
    TO = o_ref.shape[2]

    @pl.when(q == 0)
    def _():
        o_ref[0] = jnp.zeros((S, TO), jnp.float32)

    CS = S + KG + OFF - 1
    c = hp_ref[0, pl.ds(pl.multiple_of(8 * q, 8), CS), :].astype(jnp.bfloat16)
    acc = jnp.zeros((S, TO), jnp.float32)
    for r in range(KG):
        hs = c[r + OFF:r + OFF + S]
        acc = acc + jax.lax.dot_general(
            hs, wt_ref[0, r], (((1,), (1,)), ((), ())),
            preferred_element_type=jnp.float32)
    o_ref[0] += acc

    @pl.when(q == Q - 1)
    def _():
        o_ref[0] = _gelu(o_ref[0] + cb_ref[0])


def _conv_block(hp, conv_wt, cb, S, K, PADL):
    B, SP, D = hp.shape
    KPAD, O, _ = conv_wt.shape
    KG = 8
    Q = KPAD // KG
    J = 2
    TO = O // J
    OFF = PADL - K // 2
    wt4 = conv_wt.reshape(Q, KG, O, D)
    return pl.pallas_call(
        functools.partial(_conv_body, S, D, OFF, KG, Q),
        grid=(B, J, Q),
        in_specs=[
            pl.BlockSpec((1, SP, D), lambda b, j, q: (b, 0, 0)),
            pl.BlockSpec((1, KG, TO, D), lambda b, j, q: (q, 0, j, 0)),
            pl.BlockSpec((1, TO), lambda b, j, q: (0, j)),
        ],
        out_specs=pl.BlockSpec((1, S, TO), lambda b, j, q: (b, 0, j)),
        out_shape=jax.ShapeDtypeStruct((B, S, O), jnp.float32),
        interpret=_INTERPRET,
    )(hp, wt4, cb)


def _qkv_body(hc_ref, x_ref, anw_ref, anb_ref, wq_ref, qb_ref, o_ref):
    hv = hc_ref[...] + x_ref[...]
    mu = jnp.mean(hv, axis=1, keepdims=True)
    var = jnp.mean((hv - mu) ** 2, axis=1, keepdims=True)
    hn = ((hv - mu) * jax.lax.rsqrt(var + 1e-5) * anw_ref[0] + anb_ref[0])
    z = jnp.dot(hn.astype(jnp.bfloat16), wq_ref[...],
                preferred_element_type=jnp.float32) + qb_ref[0]
    o_ref[...] = z.astype(jnp.bfloat16)


def _qkv_block(hcflat, xflat, anw, anb, wq, qb, TQ):
    N, D = hcflat.shape
    D3 = wq.shape[1]
    return pl.pallas_call(
        _qkv_body,
        grid=(N // TQ,),
        in_specs=[
            pl.BlockSpec((TQ, D), lambda i: (i, 0)),
            pl.BlockSpec((TQ, D), lambda i: (i, 0)),
            pl.BlockSpec((1, D), lambda i: (0, 0)),
            pl.BlockSpec((1, D), lambda i: (0, 0)),
            pl.BlockSpec((D, D3), lambda i: (0, 0)),
            pl.BlockSpec((1, D3), lambda i: (0, 0)),
        ],
        out_specs=pl.BlockSpec((TQ, D3), lambda i: (i, 0)),
        out_shape=jax.ShapeDtypeStruct((N, D3), jnp.bfloat16),
        interpret=_INTERPRET,
    )(hcflat, xflat, anw, anb, wq, qb)


def _attn_body(scale, dh, q_ref, k_ref, v_ref, o_ref):
    for e in range(128 // dh):
        q = q_ref[0][:, e * dh:(e + 1) * dh]
        kk = k_ref[0][:, e * dh:(e + 1) * dh]
        s = jax.lax.dot_general(q, kk, (((1,), (1,)), ((), ())),
                                preferred_element_type=jnp.float32) * scale
        m = jnp.max(s, axis=1, keepdims=True)
        p = jnp.exp(s - m)
        pb = p.astype(jnp.bfloat16)
        o = jnp.dot(pb, v_ref[0][:, e * dh:(e + 1) * dh],
                    preferred_element_type=jnp.float32)
        o = o / jnp.sum(p, axis=1, keepdims=True)
        o_ref[0, :, e * dh:(e + 1) * dh] = o.astype(jnp.bfloat16)


def _attn_block(qkv, B, S, D, H):
    dh = D // H
    HB = D // 128
    scale = 1.0 / (dh ** 0.5)
    return pl.pallas_call(
        functools.partial(_attn_body, scale, dh),
        grid=(B, HB),
        in_specs=[
            pl.BlockSpec((1, S, 128), lambda b, h: (b, 0, h)),
            pl.BlockSpec((1, S, 128), lambda b, h: (b, 0, HB + h)),
            pl.BlockSpec((1, S, 128), lambda b, h: (b, 0, 2 * HB + h)),
        ],
        out_specs=pl.BlockSpec((1, S, 128), lambda b, h: (b, 0, h)),
        out_shape=jax.ShapeDtypeStruct((B, S, D), jnp.bfloat16),
        interpret=_INTERPRET,
    )(qkv, qkv, qkv)


def _oproj_body(o_ref, wo_ref, ob_ref, out_ref):
    z = jnp.dot(o_ref[...], wo_ref[...],
                preferred_element_type=jnp.float32) + ob_ref[0]
    out_ref[...] = z


def _oproj_block(oflat, wo, ob, TQ):
    N, D = oflat.shape
    return pl.pallas_call(
        _oproj_body,
        grid=(N // TQ,),
        in_specs=[
            pl.BlockSpec((TQ, D), lambda i: (i, 0)),
            pl.BlockSpec((D, D), lambda i: (0, 0)),
            pl.BlockSpec((1, D), lambda i: (0, 0)),
        ],
        out_specs=pl.BlockSpec((TQ, D), lambda i: (i, 0)),
        out_shape=jax.ShapeDtypeStruct((N, D), jnp.float32),
        interpret=_INTERPRET,
    )(oflat, wo, ob)


def _moe_body(EPG, g_ref, h_ref, hc_ref, x_ref, w1_ref, b1_ref, w2_ref,
              b2_ref, o_ref):
    g = pl.program_id(1)
    resv = hc_ref[...] + x_ref[...]
    hv = h_ref[...] + resv
    hb = hv.astype(jnp.bfloat16)
    acc = jnp.zeros(o_ref.shape, jnp.float32)
    for e in range(EPG):
        z = jnp.dot(hb, w1_ref[0, e], preferred_element_type=jnp.float32)
        z = z + b1_ref[0, e:e + 1, :]
        a = _gelu(z).astype(jnp.bfloat16)
        y = jnp.dot(a, w2_ref[0, e], preferred_element_type=jnp.float32)
        acc = acc + y + b2_ref[0, e:e + 1, :]
    go = acc * (1.0 / EPG)
    mask = g_ref[0] == g

    @pl.when(g == 0)
    def _():
        o_ref[...] = resv

    o_ref[...] = jnp.where(mask, go + resv, o_ref[...])


def _moe_block(gid3, oproj, hcflat, xflat, w1b, b1, w2b, b2, TM):
    N, D = oproj.shape
    G, EPG, _, DFF = w1b.shape
    return pl.pallas_call(
        functools.partial(_moe_body, EPG),
        grid=(N // TM, G),
        in_specs=[
            pl.BlockSpec((1, TM, 1), lambda t, g: (t, 0, 0)),
            pl.BlockSpec((TM, D), lambda t, g: (t, 0)),
            pl.BlockSpec((TM, D), lambda t, g: (t, 0)),
            pl.BlockSpec((TM, D), lambda t, g: (t, 0)),
            pl.BlockSpec((1, EPG, D, DFF), lambda t, g: (g, 0, 0, 0)),
            pl.BlockSpec((1, EPG, DFF), lambda t, g: (g, 0, 0)),
            pl.BlockSpec((1, EPG, DFF, D), lambda t, g: (g, 0, 0, 0)),
            pl.BlockSpec((1, EPG, D), lambda t, g: (g, 0, 0)),
        ],
        out_specs=pl.BlockSpec((TM, D), lambda t, g: (t, 0)),
        out_shape=jax.ShapeDtypeStruct((N, D), jnp.float32),
        interpret=_INTERPRET,
    )(gid3, oproj, hcflat, xflat, w1b, b1, w2b, b2)


def kernel(x, group_ids, conv_norm_w, conv_norm_b, conv_w, conv_b,
           attn_norm_w, attn_norm_b, in_proj_w, in_proj_b,
           out_proj_w, out_proj_b, w1, b1, w2, b2):
    B, S, D = x.shape
    K = conv_w.shape[2]
    H = 16 if D % 16 == 0 and (D // 16) % 64 == 0 else D // 64
    N = B * S
    TQ = min(512, N)
    TM = min(512, N)

    KPAD = ((K + 7) // 8) * 8
    conv_wt = jnp.pad(conv_w.transpose(2, 0, 1),
                      ((0, KPAD - K), (0, 0), (0, 0))).astype(jnp.bfloat16)
    wq = in_proj_w.T.astype(jnp.bfloat16)
    wo = out_proj_w.T.astype(jnp.bfloat16)
    w1b = w1.astype(jnp.bfloat16)
    w2b = w2.astype(jnp.bfloat16)

    cnw = conv_norm_w.reshape(1, D)
    cnb = conv_norm_b.reshape(1, D)
    cb = conv_b.reshape(1, D)
    anw = attn_norm_w.reshape(1, D)
    anb = attn_norm_b.reshape(1, D)
    qb = in_proj_b.reshape(1, 3 * D)
    ob = out_proj_b.reshape(1, D)

    PADL = 16
    SP = PADL + S + PADL
    hp = _ln_pad(x, cnw, cnb, PADL, SP)
    hconv = _conv_block(hp, conv_wt, cb, S, K, PADL)
    hcflat = hconv.reshape(N, D)
    xflat = x.reshape(N, D)
    qkv = _qkv_block(hcflat, xflat, anw, anb, wq, qb, TQ).reshape(B, S, 3 * D)
    o = _attn_block(qkv, B, S, D, H)
    oproj = _oproj_block(o.reshape(N, D), wo, ob, TQ)
    gid3 = group_ids.reshape(N, 1).reshape(N // TM, TM, 1)
    out = _moe_block(gid3, oproj, hcflat, xflat, w1b, b1, w2b, b2, TM)
    return out.reshape(B, S, D)
